# baseline (speedup 1.0000x reference)
import jax
import jax.numpy as jnp
from jax.experimental import pallas as pl


def _dyn_graph_wave_kernel(x_ref, n1_ref, wself_ref, wagg_ref, out_ref):
    N = n1_ref.shape[0]
    n, F = x_ref.shape
    B = n // N
    n1 = n1_ref[...]
    logits = jax.lax.dot_general(
        n1, n1, (((1,), (1,)), ((), ())), preferred_element_type=jnp.float32
    )
    s = jax.nn.sigmoid(logits)
    w = jnp.where(s > 0.5, s, 0.0)
    xp = jnp.concatenate([x_ref[b * N:(b + 1) * N, :] for b in range(B)], axis=1)
    agg = jax.lax.dot_general(
        w, xp, (((0,), (0,)), ((), ())), preferred_element_type=jnp.float32
    )
    bi = jax.lax.broadcasted_iota(jnp.int32, (B * F, B * F), 0) // F
    bj = jax.lax.broadcasted_iota(jnp.int32, (B * F, B * F), 1) // F
    blk = (bi == bj).astype(jnp.float32)
    wagg_blk = jnp.tile(wagg_ref[...], (B, B)) * blk
    wself_blk = jnp.tile(wself_ref[...], (B, B)) * blk
    out_p = (
        jax.lax.dot_general(
            agg, wagg_blk, (((1,), (0,)), ((), ())),
            preferred_element_type=jnp.float32,
        )
        + jax.lax.dot_general(
            xp, wself_blk, (((1,), (0,)), ((), ())),
            preferred_element_type=jnp.float32,
        )
    )
    for b in range(B):
        out_ref[b * N:(b + 1) * N, :] = out_p[:, b * F:(b + 1) * F]


def kernel(x, ptr, node1, W_self, W_agg):
    del ptr
    return pl.pallas_call(
        _dyn_graph_wave_kernel,
        out_shape=jax.ShapeDtypeStruct(x.shape, x.dtype),
    )(x, node1, W_self, W_agg)


# speedup vs baseline: 1.4960x; 1.4960x over previous
import jax
import jax.numpy as jnp
from jax.experimental import pallas as pl


def _dyn_graph_wave_kernel(n1_ref, xp_ref, wself_ref, wagg_ref, out_ref):
    N = n1_ref.shape[0]
    BF = xp_ref.shape[1]
    F = wself_ref.shape[0]
    B = BF // F
    n1 = n1_ref[...]
    logits = jax.lax.dot_general(
        n1, n1, (((1,), (1,)), ((), ())), preferred_element_type=jnp.float32
    )
    s = jax.nn.sigmoid(logits)
    w = jnp.where(s > 0.5, s, 0.0)
    xp = xp_ref[...]
    agg = jax.lax.dot_general(
        w, xp, (((0,), (0,)), ((), ())), preferred_element_type=jnp.float32
    )
    bi = jax.lax.broadcasted_iota(jnp.int32, (BF, BF), 0) // F
    bj = jax.lax.broadcasted_iota(jnp.int32, (BF, BF), 1) // F
    blk = (bi == bj).astype(jnp.float32)
    wagg_blk = jnp.tile(wagg_ref[...], (B, B)) * blk
    wself_blk = jnp.tile(wself_ref[...], (B, B)) * blk
    out_ref[...] = (
        jax.lax.dot_general(
            agg, wagg_blk, (((1,), (0,)), ((), ())),
            preferred_element_type=jnp.float32,
        )
        + jax.lax.dot_general(
            xp, wself_blk, (((1,), (0,)), ((), ())),
            preferred_element_type=jnp.float32,
        )
    )


def kernel(x, ptr, node1, W_self, W_agg):
    del ptr
    N, _ = node1.shape
    n, F = x.shape
    B = n // N
    xp = x.reshape(B, N, F).transpose(1, 0, 2).reshape(N, B * F)
    out_p = pl.pallas_call(
        _dyn_graph_wave_kernel,
        out_shape=jax.ShapeDtypeStruct((N, B * F), x.dtype),
    )(node1, xp, W_self, W_agg)
    return out_p.reshape(N, B, F).transpose(1, 0, 2).reshape(n, F)
